# baseline (speedup 1.0000x reference)
import jax
import jax.numpy as jnp
from jax.experimental import pallas as pl


def _transpose_body(x_ref, o_ref):
    eye = jnp.eye(x_ref.shape[0], dtype=x_ref.dtype)
    o_ref[...] = jax.lax.dot_general(
        x_ref[...], eye, (((0,), (0,)), ((), ())),
        preferred_element_type=jnp.float32,
    )


def kernel(input):
    bs, C, H, W = input.shape
    A = 3
    attrs = C // A
    HW = H * W
    x = input.reshape(bs * A, attrs, HW)

    out = pl.pallas_call(
        _transpose_body,
        grid=(bs * A,),
        in_specs=[pl.BlockSpec((None, attrs, HW), lambda i: (i, 0, 0))],
        out_specs=pl.BlockSpec((None, HW, attrs), lambda i: (i, 0, 0)),
        out_shape=jax.ShapeDtypeStruct((bs * A, HW, attrs), x.dtype),
    )(x)
    return out.reshape(bs, A, H, W, attrs)


# speedup vs baseline: 1.9014x; 1.9014x over previous
import jax
import jax.numpy as jnp
from jax.experimental import pallas as pl


def _transpose_body(x_ref, o_ref):
    o_ref[...] = jnp.transpose(x_ref[...], (1, 2, 0))


def kernel(input):
    bs, C, H, W = input.shape
    A = 3
    attrs = C // A

    return pl.pallas_call(
        _transpose_body,
        grid=(bs, A),
        in_specs=[
            pl.BlockSpec((None, attrs, H, W), lambda b, a: (b, a, 0, 0))
        ],
        out_specs=pl.BlockSpec(
            (None, None, H, W, attrs), lambda b, a: (b, a, 0, 0, 0)
        ),
        out_shape=jax.ShapeDtypeStruct((bs, A, H, W, attrs), input.dtype),
    )(input)
